# Initial kernel scaffold; baseline (speedup 1.0000x reference)
import jax
import jax.numpy as jnp
from jax.experimental import pallas as pl


def kernel(x, edge_index, edge_attr, batch, W0, b0, W1, b1, W2, b2, Wf, bf):
    raise NotImplementedError("write your pallas kernel here")


# speedup vs baseline: 3.1298x; 3.1298x over previous
import functools

import jax
import jax.numpy as jnp
from jax import lax
from jax.experimental import pallas as pl
from jax.experimental.pallas import tpu as pltpu
from jax.experimental.pallas import tpu_sc as plsc

N_NODES = 10000
N_PAD = 10240
E_PAD = 163840
ROWS_PER_TILE = 640
BN = 256
GRID = N_PAD // BN

_MESH = plsc.VectorSubcoreMesh(core_axis_name="c", subcore_axis_name="s")


def _make_hop(CH):
    @functools.partial(
        pl.kernel,
        out_type=jax.ShapeDtypeStruct((2, N_PAD, 128), jnp.float32),
        mesh=_MESH,
        scratch_types=[
            pltpu.VMEM((CH, 128), jnp.int32),
            pltpu.VMEM((CH, 128), jnp.int32),
            pltpu.VMEM((128, 128), jnp.float32),
            pltpu.VMEM_SHARED((N_PAD, 128), jnp.float32),
            pltpu.SemaphoreType.DMA,
        ],
    )
    def hop(t_hbm, rows_hbm, cols_hbm, zero_hbm, out_hbm,
            row_v, col_v, gbuf, acc, sem):
        c = lax.axis_index("c")
        s = lax.axis_index("s")
        pltpu.sync_copy(rows_hbm.at[c, s], row_v)
        pltpu.sync_copy(cols_hbm.at[c, s], col_v)
        pltpu.sync_copy(zero_hbm, acc.at[pl.ds(s * ROWS_PER_TILE, ROWS_PER_TILE)])
        plsc.subcore_barrier()

        def body(j, carry):
            pltpu.async_copy(t_hbm.at[row_v.at[j]], gbuf, sem).wait()
            pltpu.sync_copy(gbuf, acc.at[col_v.at[j]], add=True)
            return carry

        lax.fori_loop(0, CH, body, 0)
        plsc.subcore_barrier()
        pltpu.sync_copy(
            acc.at[pl.ds(s * ROWS_PER_TILE, ROWS_PER_TILE)],
            out_hbm.at[c, pl.ds(s * ROWS_PER_TILE, ROWS_PER_TILE)],
        )

    return hop


_hop_wide = _make_hop(80)
_hop_half = _make_hop(40)


@functools.partial(
    pl.kernel,
    out_type=jax.ShapeDtypeStruct((2, N_PAD, 128), jnp.float32),
    mesh=_MESH,
    scratch_types=[
        pltpu.VMEM((40, 128), jnp.int32),
        pltpu.VMEM((128, 128), jnp.float32),
        pltpu.VMEM_SHARED((N_PAD, 128), jnp.float32),
    ],
)
def _deg_hop(ones_hbm, cols_hbm, zero_hbm, out_hbm, col_v, obuf, acc):
    c = lax.axis_index("c")
    s = lax.axis_index("s")
    pltpu.sync_copy(cols_hbm.at[c, s], col_v)
    pltpu.sync_copy(ones_hbm, obuf)
    pltpu.sync_copy(zero_hbm, acc.at[pl.ds(s * ROWS_PER_TILE, ROWS_PER_TILE)])
    plsc.subcore_barrier()

    def body(j, carry):
        pltpu.sync_copy(obuf, acc.at[col_v.at[j]], add=True)
        return carry

    lax.fori_loop(0, 40, body, 0)
    plsc.subcore_barrier()
    pltpu.sync_copy(
        acc.at[pl.ds(s * ROWS_PER_TILE, ROWS_PER_TILE)],
        out_hbm.at[c, pl.ds(s * ROWS_PER_TILE, ROWS_PER_TILE)],
    )


def _prep_kernel(degs_ref, x_ref, t0_ref, dinv_ref, dinv2_ref):
    deg = degs_ref[0, :, 0:1] + degs_ref[1, :, 0:1]
    dinv = jnp.where(deg > 0, lax.rsqrt(jnp.maximum(deg, 1e-12)), 0.0)
    xs = x_ref[...] * dinv
    t0_ref[...] = jnp.concatenate(
        [xs, jnp.zeros((BN, 120), jnp.float32)], axis=1)
    dinv_ref[...] = jnp.broadcast_to(dinv, (BN, 8))
    dinv2_ref[...] = jnp.broadcast_to(dinv * dinv, (BN, 8))


def _prep(degs, xp):
    return pl.pallas_call(
        _prep_kernel,
        grid=(GRID,),
        in_specs=[
            pl.BlockSpec((2, BN, 128), lambda i: (0, i, 0)),
            pl.BlockSpec((BN, 8), lambda i: (i, 0)),
        ],
        out_specs=[
            pl.BlockSpec((BN, 128), lambda i: (i, 0)),
            pl.BlockSpec((BN, 8), lambda i: (i, 0)),
            pl.BlockSpec((BN, 8), lambda i: (i, 0)),
        ],
        out_shape=[
            jax.ShapeDtypeStruct((N_PAD, 128), jnp.float32),
            jax.ShapeDtypeStruct((N_PAD, 8), jnp.float32),
            jax.ShapeDtypeStruct((N_PAD, 8), jnp.float32),
        ],
    )(degs, xp)


def _combine_half_kernel(s_ref, dinv2_ref, t_ref):
    t_ref[...] = (s_ref[0] + s_ref[1]) * dinv2_ref[:, 0:1]


def _combine_half(s, dinv2):
    return pl.pallas_call(
        _combine_half_kernel,
        grid=(GRID,),
        in_specs=[
            pl.BlockSpec((2, BN, 128), lambda i: (0, i, 0)),
            pl.BlockSpec((BN, 8), lambda i: (i, 0)),
        ],
        out_specs=pl.BlockSpec((BN, 128), lambda i: (i, 0)),
        out_shape=jax.ShapeDtypeStruct((N_PAD, 128), jnp.float32),
    )(s, dinv2)


def _scale_wide_kernel(s_ref, dinv2_ref, t_ref):
    t_ref[...] = s_ref[...] * dinv2_ref[:, 0:1][None]


def _scale_wide(s, dinv2):
    return pl.pallas_call(
        _scale_wide_kernel,
        grid=(GRID,),
        in_specs=[
            pl.BlockSpec((2, BN, 128), lambda i: (0, i, 0)),
            pl.BlockSpec((BN, 8), lambda i: (i, 0)),
        ],
        out_specs=pl.BlockSpec((2, BN, 128), lambda i: (0, i, 0)),
        out_shape=jax.ShapeDtypeStruct((2, N_PAD, 128), jnp.float32),
    )(s, dinv2)


def _split2(o):
    return jnp.stack([o[:, :128], o[:, 128:]])


def _layer0_mm_kernel(xp_ref, s1_ref, s2_ref, s3_ref, dinv_ref, w_ref, b_ref,
                      h_ref, t_ref):
    dinv = dinv_ref[:, 0:1]
    feats = jnp.concatenate(
        [xp_ref[...]]
        + [(s_ref[0, :, :8] + s_ref[1, :, :8]) * dinv
           for s_ref in (s1_ref, s2_ref, s3_ref)],
        axis=1,
    )
    o = jnp.dot(feats, w_ref[...], preferred_element_type=jnp.float32)
    o = jnp.maximum(o + b_ref[...], 0.0)
    h_ref[...] = _split2(o)
    t_ref[...] = _split2(o * dinv)


def _layer0_mm(xp, s1, s2, s3, dinv, w, b):
    spec_w = pl.BlockSpec((2, BN, 128), lambda i: (0, i, 0))
    return pl.pallas_call(
        _layer0_mm_kernel,
        grid=(GRID,),
        in_specs=[
            pl.BlockSpec((BN, 8), lambda i: (i, 0)),
            spec_w, spec_w, spec_w,
            pl.BlockSpec((BN, 8), lambda i: (i, 0)),
            pl.BlockSpec((32, 256), lambda i: (0, 0)),
            pl.BlockSpec((1, 256), lambda i: (0, 0)),
        ],
        out_specs=[spec_w, spec_w],
        out_shape=[
            jax.ShapeDtypeStruct((2, N_PAD, 128), jnp.float32),
            jax.ShapeDtypeStruct((2, N_PAD, 128), jnp.float32),
        ],
    )(xp, s1, s2, s3, dinv, w, b)


def _cat2(p_ref):
    return jnp.concatenate([p_ref[0], p_ref[1]], axis=1)


def _layer_mm_kernel(h_ref, s1_ref, s2_ref, s3_ref, dinv_ref, w_ref, b_ref,
                     ho_ref, to_ref):
    dinv = dinv_ref[:, 0:1]
    feats = jnp.concatenate(
        [_cat2(h_ref)] + [_cat2(s) * dinv for s in (s1_ref, s2_ref, s3_ref)],
        axis=1,
    )
    o = jnp.dot(feats, w_ref[...], preferred_element_type=jnp.float32)
    o = jnp.maximum(o + b_ref[...], 0.0)
    ho_ref[...] = _split2(o)
    to_ref[...] = _split2(o * dinv)


def _layer_mm(h, s1, s2, s3, dinv, w, b):
    spec_w = pl.BlockSpec((2, BN, 128), lambda i: (0, i, 0))
    return pl.pallas_call(
        _layer_mm_kernel,
        grid=(GRID,),
        in_specs=[
            spec_w, spec_w, spec_w, spec_w,
            pl.BlockSpec((BN, 8), lambda i: (i, 0)),
            pl.BlockSpec((1024, 256), lambda i: (0, 0)),
            pl.BlockSpec((1, 256), lambda i: (0, 0)),
        ],
        out_specs=[spec_w, spec_w],
        out_shape=[
            jax.ShapeDtypeStruct((2, N_PAD, 128), jnp.float32),
            jax.ShapeDtypeStruct((2, N_PAD, 128), jnp.float32),
        ],
    )(h, s1, s2, s3, dinv, w, b)


def _final_mm_kernel(h_ref, s1_ref, s2_ref, s3_ref, dinv_ref, w_ref, b_ref,
                     wf_ref, bf_ref, out_ref):
    dinv = dinv_ref[:, 0:1]
    feats = jnp.concatenate(
        [_cat2(h_ref)] + [_cat2(s) * dinv for s in (s1_ref, s2_ref, s3_ref)],
        axis=1,
    )
    o = jnp.dot(feats, w_ref[...], preferred_element_type=jnp.float32)
    o = jnp.maximum(o + b_ref[...], 0.0)
    z = jnp.dot(o, wf_ref[...], preferred_element_type=jnp.float32)
    out_ref[...] = jax.nn.sigmoid(z + bf_ref[...])


def _final_mm(h, s1, s2, s3, dinv, w, b, wf, bf):
    spec_w = pl.BlockSpec((2, BN, 128), lambda i: (0, i, 0))
    return pl.pallas_call(
        _final_mm_kernel,
        grid=(GRID,),
        in_specs=[
            spec_w, spec_w, spec_w, spec_w,
            pl.BlockSpec((BN, 8), lambda i: (i, 0)),
            pl.BlockSpec((1024, 256), lambda i: (0, 0)),
            pl.BlockSpec((1, 256), lambda i: (0, 0)),
            pl.BlockSpec((256, 1), lambda i: (0, 0)),
            pl.BlockSpec((1, 1), lambda i: (0, 0)),
        ],
        out_specs=pl.BlockSpec((BN, 1), lambda i: (i, 0)),
        out_shape=jax.ShapeDtypeStruct((N_PAD, 1), jnp.float32),
    )(h, s1, s2, s3, dinv, w, b, wf, bf)


@jax.jit
def kernel(x, edge_index, edge_attr, batch, W0, b0, W1, b1, W2, b2, Wf, bf):
    del edge_attr, batch
    E = edge_index.shape[1]
    n_extra = E_PAD - E
    rows = jnp.concatenate([edge_index[0],
                            jnp.zeros((n_extra,), jnp.int32)])
    cols = jnp.concatenate([edge_index[1],
                            jnp.full((n_extra,), N_NODES, jnp.int32)])

    rows_w16 = rows.reshape(16, 80, 128)
    rows_w = jnp.stack([rows_w16, rows_w16 + N_PAD])
    cols_w16 = cols.reshape(16, 80, 128)
    cols_w = jnp.stack([cols_w16, cols_w16])
    rows_n = rows.reshape(2, 16, 40, 128)
    cols_n = cols.reshape(2, 16, 40, 128)

    zero_w = jnp.zeros((ROWS_PER_TILE, 128), jnp.float32)
    ones_t = jnp.ones((128, 128), jnp.float32)
    xp = jnp.zeros((N_PAD, 8), jnp.float32).at[:N_NODES].set(x)

    def hop_h(t):
        return _hop_half(t, rows_n, cols_n, zero_w)

    def hop_w(t):
        return _hop_wide(t.reshape(2 * N_PAD, 128), rows_w, cols_w, zero_w)

    degs = _deg_hop(ones_t, cols_n, zero_w)
    t0, dinv, dinv2 = _prep(degs, xp)

    s1 = hop_h(t0)
    s2 = hop_h(_combine_half(s1, dinv2))
    s3 = hop_h(_combine_half(s2, dinv2))
    h, t = _layer0_mm(xp, s1, s2, s3, dinv, W0.reshape(32, 256),
                      b0.reshape(1, 256))

    s1 = hop_w(t)
    s2 = hop_w(_scale_wide(s1, dinv2))
    s3 = hop_w(_scale_wide(s2, dinv2))
    h, t = _layer_mm(h, s1, s2, s3, dinv, W1.reshape(1024, 256),
                     b1.reshape(1, 256))

    s1 = hop_w(t)
    s2 = hop_w(_scale_wide(s1, dinv2))
    s3 = hop_w(_scale_wide(s2, dinv2))
    out = _final_mm(h, s1, s2, s3, dinv, W2.reshape(1024, 256),
                    b2.reshape(1, 256), Wf, bf.reshape(1, 1))
    return out[:N_NODES]
